# baseline (speedup 1.0000x reference)
import functools

import jax
import jax.numpy as jnp
from jax import lax
from jax.experimental import pallas as pl
from jax.experimental.pallas import tpu as pltpu
from jax.experimental.pallas import tpu_sc as plsc

N = 10000
E = 320000
G = 64

_NC, _NS = 2, 16
_NW = _NC * _NS
_DISCARD = N

_E_PAD = 327680
_DC_IDX = 331776
_CNT_OFF = 10240
_DC_ROWS = 10368
_ACC_ROWS = 10240
_OUT_ROWS = 10112
_POOL_IDX = 12288
_BR = 400


def _sc_gather_scatter_add(n_idx, F, acc_rows, out_rows, chunk,
                           const_ones=False, split=None, pack_didx=False):
    total_chunks = n_idx // chunk
    assert n_idx % (_NW * chunk) == 0
    if split is None:
        nc0 = nc1 = total_chunks // _NW
    else:
        nc0, nc1 = split
        assert _NS * (nc0 + nc1) == total_chunks
        assert nc0 % 8 == 0 and nc1 % 8 == 0
    nc_max = max(nc0, nc1)
    static_nc = nc0 == nc1
    zrows = acc_rows // _NS
    orows = out_rows // _NS
    nzfull, zrem = divmod(zrows, chunk)
    mesh = plsc.VectorSubcoreMesh(core_axis_name="c", subcore_axis_name="s")
    assert const_ones or (nc0 % 2 == 0 and nc1 % 2 == 0)

    @functools.partial(
        pl.kernel,
        out_type=jax.ShapeDtypeStruct((_NC, out_rows, F), jnp.float32),
        mesh=mesh,
        scratch_types=[
            pltpu.VMEM((chunk,), jnp.int32),
            pltpu.VMEM((chunk,), jnp.int32),
            pltpu.VMEM((nc_max, chunk // 2 if pack_didx else chunk),
                       jnp.int32),
            pltpu.VMEM((chunk,), jnp.int32),
            pltpu.VMEM((chunk, F), jnp.float32),
            pltpu.VMEM((chunk, F), jnp.float32),
            pltpu.VMEM_SHARED((acc_rows, F), jnp.float32),
            pltpu.SemaphoreType.DMA,
            pltpu.SemaphoreType.DMA,
        ],
        compiler_params=pltpu.CompilerParams(use_tc_tiling_on_sc=False),
    )
    def body(g_hbm, src_hbm, dst_hbm, out_hbm,
             sidx0_v, sidx1_v, didx_v, dst_st_v, rows0_v, rows1_v, acc_sh,
             sem_a, sem_b):
        cid = lax.axis_index("c")
        sid = lax.axis_index("s")

        if static_nc:
            nc_w = nc0
            cb = (cid * _NS + sid) * nc0
        else:
            nc_w = jnp.where(cid == 0, nc0, nc1)
            cb = jnp.where(cid == 0, sid * nc0, _NS * nc0 + sid * nc1)

        def work():
            pltpu.sync_copy(dst_hbm.at[pl.ds(cb, nc_max), :], didx_v)

            if pack_didx:
                def dst_idx(i):
                    for k in range(chunk // 32):
                        w = didx_v[i, pl.ds(k * 16, 16)]
                        dst_st_v[pl.ds(k * 16, 16)] = jnp.bitwise_and(
                            w, 0xFFFF)
                        dst_st_v[pl.ds(chunk // 2 + k * 16, 16)] = (
                            lax.shift_right_logical(w, 16))
                    return dst_st_v
            else:
                def dst_idx(i):
                    return didx_v.at[i]

            def zrow(r, carry):
                for j in range(F // 16):
                    rows0_v[r, pl.ds(j * 16, 16)] = jnp.zeros(
                        (16,), jnp.float32)
                return carry
            lax.fori_loop(0, chunk, zrow, 0)
            zbase = sid * zrows
            for t in range(nzfull):
                pltpu.sync_copy(rows0_v,
                                acc_sh.at[pl.ds(zbase + t * chunk, chunk), :])
            if zrem:
                pltpu.sync_copy(
                    rows0_v.at[pl.ds(0, zrem), :],
                    acc_sh.at[pl.ds(zbase + nzfull * chunk, zrem), :])
            if const_ones:
                def orow(r, carry):
                    for j in range(F // 16):
                        rows0_v[r, pl.ds(j * 16, 16)] = jnp.ones(
                            (16,), jnp.float32)
                    return carry
                lax.fori_loop(0, chunk, orow, 0)
            plsc.subcore_barrier()

            if const_ones:
                def step(i, carry):
                    pltpu.sync_copy(rows0_v, acc_sh.at[dst_idx(i)],
                                    add=True)
                    return carry
                lax.fori_loop(0, nc_w, step, 0)
            else:
                base = cb * chunk

                def sload(i, buf):
                    pltpu.sync_copy(
                        src_hbm.at[pl.ds(base + i * chunk, chunk)], buf)

                def gather(idx_v, rows_v, sem):
                    pltpu.async_copy(g_hbm.at[idx_v], rows_v, sem)

                def gwait(rows_v, sem):
                    pltpu.make_async_copy(g_hbm.at[sidx0_v], rows_v,
                                          sem).wait()

                sload(0, sidx0_v)
                gather(sidx0_v, rows0_v, sem_a)
                sload(1, sidx1_v)

                def pair(j, carry):
                    i0 = 2 * j
                    gather(sidx1_v, rows1_v, sem_b)
                    d0 = dst_idx(i0)
                    gwait(rows0_v, sem_a)
                    sload(i0 + 2, sidx0_v)
                    pltpu.sync_copy(rows0_v, acc_sh.at[d0], add=True)
                    gather(sidx0_v, rows0_v, sem_a)
                    d1 = dst_idx(i0 + 1)
                    gwait(rows1_v, sem_b)
                    sload(i0 + 3, sidx1_v)
                    pltpu.sync_copy(rows1_v, acc_sh.at[d1], add=True)
                    return carry
                lax.fori_loop(0, nc_w // 2, pair, 0)
                gwait(rows0_v, sem_a)

            plsc.subcore_barrier()
            pltpu.sync_copy(acc_sh.at[pl.ds(sid * orows, orows), :],
                            out_hbm.at[cid, pl.ds(sid * orows, orows), :])

        work()

    return body


_sc_degcnt = _sc_gather_scatter_add(_DC_IDX, 16, _DC_ROWS, _DC_ROWS, 128,
                                    const_ones=True)
_sc_aggr = {
    128: _sc_gather_scatter_add(_E_PAD, 128, _OUT_ROWS, _OUT_ROWS, 128,
                                split=(136, 24), pack_didx=True),
    64: _sc_gather_scatter_add(_E_PAD, 64, _OUT_ROWS, _OUT_ROWS, 128,
                               split=(136, 24), pack_didx=True),
    32: _sc_gather_scatter_add(_E_PAD, 32, _OUT_ROWS, _OUT_ROWS, 128,
                               split=(120, 40)),
    16: _sc_gather_scatter_add(_E_PAD, 16, _OUT_ROWS, _OUT_ROWS, 128,
                               split=(112, 48)),
}
_sc_pool = _sc_gather_scatter_add(_POOL_IDX, 16, 128, 128, 64)


def _tc_dis(d0, d1):
    def body(a_ref, b_ref, o_ref):
        o_ref[...] = lax.rsqrt(a_ref[..., :1] + b_ref[..., :1] + 1.0)
    return pl.pallas_call(
        body,
        grid=(N // _BR,),
        in_specs=[pl.BlockSpec((_BR, 16), lambda i: (i, 0)),
                  pl.BlockSpec((_BR, 16), lambda i: (i, 0))],
        out_specs=pl.BlockSpec((_BR, 1), lambda i: (i, 0)),
        out_shape=jax.ShapeDtypeStruct((N, 1), jnp.float32),
    )(d0, d1)


def _tc_first(x, W, dis):
    F_in, F_out = W.shape

    def body(x_ref, w_ref, d_ref, o_ref):
        o_ref[...] = d_ref[...] * jnp.dot(
            x_ref[...], w_ref[...], preferred_element_type=jnp.float32)
    return pl.pallas_call(
        body,
        grid=(N // _BR,),
        in_specs=[pl.BlockSpec((_BR, F_in), lambda i: (i, 0)),
                  pl.BlockSpec((F_in, F_out), lambda i: (0, 0)),
                  pl.BlockSpec((_BR, 1), lambda i: (i, 0))],
        out_specs=pl.BlockSpec((_BR, F_out), lambda i: (i, 0)),
        out_shape=jax.ShapeDtypeStruct((N, F_out), jnp.float32),
    )(x, W, dis)


def _tc_mid(p0, p1, g, dis, b, W):
    F_in, F_out = W.shape

    def body(p0_ref, p1_ref, g_ref, d_ref, b_ref, w_ref, o_ref):
        a = jnp.maximum(
            d_ref[...] * (p0_ref[...] + p1_ref[...] + g_ref[...]) + b_ref[...],
            0.0)
        o_ref[...] = d_ref[...] * jnp.dot(
            a, w_ref[...], preferred_element_type=jnp.float32)
    return pl.pallas_call(
        body,
        grid=(N // _BR,),
        in_specs=[pl.BlockSpec((_BR, F_in), lambda i: (i, 0)),
                  pl.BlockSpec((_BR, F_in), lambda i: (i, 0)),
                  pl.BlockSpec((_BR, F_in), lambda i: (i, 0)),
                  pl.BlockSpec((_BR, 1), lambda i: (i, 0)),
                  pl.BlockSpec((1, F_in), lambda i: (0, 0)),
                  pl.BlockSpec((F_in, F_out), lambda i: (0, 0))],
        out_specs=pl.BlockSpec((_BR, F_out), lambda i: (i, 0)),
        out_shape=jax.ShapeDtypeStruct((N, F_out), jnp.float32),
    )(p0, p1, g, dis, b, W)


def _tc_mid1(p, g, dis, b, W):
    F_in, F_out = W.shape

    def body(p_ref, g_ref, d_ref, b_ref, w_ref, o_ref):
        a = jnp.maximum(
            d_ref[...] * (p_ref[...] + g_ref[...]) + b_ref[...], 0.0)
        o_ref[...] = d_ref[...] * jnp.dot(
            a, w_ref[...], preferred_element_type=jnp.float32)
    return pl.pallas_call(
        body,
        grid=(N // _BR,),
        in_specs=[pl.BlockSpec((_BR, F_in), lambda i: (i, 0)),
                  pl.BlockSpec((_BR, F_in), lambda i: (i, 0)),
                  pl.BlockSpec((_BR, 1), lambda i: (i, 0)),
                  pl.BlockSpec((1, F_in), lambda i: (0, 0)),
                  pl.BlockSpec((F_in, F_out), lambda i: (0, 0))],
        out_specs=pl.BlockSpec((_BR, F_out), lambda i: (i, 0)),
        out_shape=jax.ShapeDtypeStruct((N, F_out), jnp.float32),
    )(p, g, dis, b, W)


def _tc_post(p0, p1, g, dis, b):
    F = g.shape[1]

    def body(p0_ref, p1_ref, g_ref, d_ref, b_ref, o_ref):
        o_ref[...] = jnp.maximum(
            d_ref[...] * (p0_ref[...] + p1_ref[...] + g_ref[...]) + b_ref[...],
            0.0)
    return pl.pallas_call(
        body,
        grid=(N // _BR,),
        in_specs=[pl.BlockSpec((_BR, F), lambda i: (i, 0)),
                  pl.BlockSpec((_BR, F), lambda i: (i, 0)),
                  pl.BlockSpec((_BR, F), lambda i: (i, 0)),
                  pl.BlockSpec((_BR, 1), lambda i: (i, 0)),
                  pl.BlockSpec((1, F), lambda i: (0, 0))],
        out_specs=pl.BlockSpec((_BR, F), lambda i: (i, 0)),
        out_shape=jax.ShapeDtypeStruct((N, F), jnp.float32),
    )(p0, p1, g, dis, b)


def _tc_head(s0, s1, c0, c1, Wp, bp):
    def body(s0_ref, s1_ref, c0_ref, c1_ref, w_ref, b_ref, o_ref):
        s = s0_ref[...] + s1_ref[...]
        cnt = jnp.maximum(c0_ref[...] + c1_ref[...], 1.0)
        o_ref[...] = jnp.dot(
            s, w_ref[...], preferred_element_type=jnp.float32) / cnt + b_ref[...]
    return pl.pallas_call(
        body,
        out_shape=jax.ShapeDtypeStruct((G, Wp.shape[1]), jnp.float32),
    )(s0, s1, c0, c1, Wp, bp)


def kernel(x, edges, batch, W1, b1, W2, b2, W3, b3, W4, b4, Wp, bp):
    src, dst = edges[0], edges[1]

    ones16 = jnp.ones((8, 16), jnp.float32)
    dc_src = jnp.zeros((8,), jnp.int32)
    dc_dst = jnp.concatenate([
        dst,
        batch + _CNT_OFF,
        jnp.full((_DC_IDX - E - N,), _DISCARD, jnp.int32),
    ]).reshape(-1, 128)
    dc = _sc_degcnt(ones16, dc_src, dc_dst)
    dis = _tc_dis(dc[0], dc[1])

    e_src = jnp.concatenate([src, jnp.zeros((_E_PAD + 512 - E,), jnp.int32)])
    e_dst128 = jnp.concatenate([
        dst, jnp.full((_E_PAD + 256 * 128 - E,), _DISCARD, jnp.int32),
    ]).reshape(-1, 128)
    e_dst_packed = e_dst128[:, :64] | (e_dst128[:, 64:] << 16)

    g1 = _tc_first(x, W1, dis)
    a1 = _sc_aggr[128](g1, e_src, e_dst_packed)
    g2 = _tc_mid(a1[0], a1[1], g1, dis, b1.reshape(1, -1), W2)
    a2 = _sc_aggr[64](g2, e_src, e_dst_packed)
    g3 = _tc_mid(a2[0], a2[1], g2, dis, b2.reshape(1, -1), W3)
    a3 = _sc_aggr[32](g3, e_src, e_dst128)
    g4 = _tc_mid(a3[0], a3[1], g3, dis, b3.reshape(1, -1), W4)
    a4 = _sc_aggr[16](g4, e_src, e_dst128)
    h4 = _tc_post(a4[0], a4[1], g4, dis, b4.reshape(1, -1))

    p_src = jnp.concatenate([
        jnp.arange(N, dtype=jnp.int32),
        jnp.zeros((_POOL_IDX + 128 - N,), jnp.int32),
    ])
    p_dst = jnp.concatenate([
        batch,
        jnp.full((_POOL_IDX - N,), G, jnp.int32),
    ]).reshape(-1, 64)
    pooled = _sc_pool(h4, p_src, p_dst)

    c0 = dc[0, _CNT_OFF:_CNT_OFF + G, :1]
    c1 = dc[1, _CNT_OFF:_CNT_OFF + G, :1]
    return _tc_head(pooled[0, :G], pooled[1, :G], c0, c1, Wp,
                    bp.reshape(1, -1))


# speedup vs baseline: 1.0335x; 1.0335x over previous
import functools

import jax
import jax.numpy as jnp
from jax import lax
from jax.experimental import pallas as pl
from jax.experimental.pallas import tpu as pltpu
from jax.experimental.pallas import tpu_sc as plsc

N = 10000
E = 320000
G = 64

_NC, _NS = 2, 16
_NW = _NC * _NS
_DISCARD = N

_E_PAD = 327680
_DC_IDX = 331776
_CNT_OFF = 10240
_DC_ROWS = 10368
_ACC_ROWS = 10240
_OUT_ROWS = 10112
_POOL_IDX = 12288
_BR = 400


def _sc_gather_scatter_add(n_idx, F, acc_rows, out_rows, chunk,
                           const_ones=False, split=None, pack_didx=False):
    total_chunks = n_idx // chunk
    assert n_idx % (_NW * chunk) == 0
    if split is None:
        nc0 = nc1 = total_chunks // _NW
    else:
        nc0, nc1 = split
        assert _NS * (nc0 + nc1) == total_chunks
        assert nc0 % 8 == 0 and nc1 % 8 == 0
    nc_max = max(nc0, nc1)
    static_nc = nc0 == nc1
    zrows = acc_rows // _NS
    orows = out_rows // _NS
    nzfull, zrem = divmod(zrows, chunk)
    mesh = plsc.VectorSubcoreMesh(core_axis_name="c", subcore_axis_name="s")
    assert const_ones or (nc0 % 2 == 0 and nc1 % 2 == 0)

    @functools.partial(
        pl.kernel,
        out_type=jax.ShapeDtypeStruct((_NC, out_rows, F), jnp.float32),
        mesh=mesh,
        scratch_types=[
            pltpu.VMEM((chunk,), jnp.int32),
            pltpu.VMEM((chunk,), jnp.int32),
            pltpu.VMEM((nc_max, chunk // 2 if pack_didx else chunk),
                       jnp.int32),
            pltpu.VMEM((chunk,), jnp.int32),
            pltpu.VMEM((chunk, F), jnp.float32),
            pltpu.VMEM((chunk, F), jnp.float32),
            pltpu.VMEM_SHARED((acc_rows, F), jnp.float32),
            pltpu.SemaphoreType.DMA,
            pltpu.SemaphoreType.DMA,
        ],
        compiler_params=pltpu.CompilerParams(use_tc_tiling_on_sc=False),
    )
    def body(g_hbm, src_hbm, dst_hbm, out_hbm,
             sidx0_v, sidx1_v, didx_v, dst_st_v, rows0_v, rows1_v, acc_sh,
             sem_a, sem_b):
        cid = lax.axis_index("c")
        sid = lax.axis_index("s")

        if static_nc:
            nc_w = nc0
            cb = (cid * _NS + sid) * nc0
        else:
            nc_w = jnp.where(cid == 0, nc0, nc1)
            cb = jnp.where(cid == 0, sid * nc0, _NS * nc0 + sid * nc1)

        def work():
            pltpu.sync_copy(dst_hbm.at[pl.ds(cb, nc_max), :], didx_v)

            if pack_didx:
                def dst_idx(i):
                    for k in range(chunk // 32):
                        w = didx_v[i, pl.ds(k * 16, 16)]
                        dst_st_v[pl.ds(k * 16, 16)] = jnp.bitwise_and(
                            w, 0xFFFF)
                        dst_st_v[pl.ds(chunk // 2 + k * 16, 16)] = (
                            lax.shift_right_logical(w, 16))
                    return dst_st_v
            else:
                def dst_idx(i):
                    return didx_v.at[i]

            def zrow(r, carry):
                for j in range(F // 16):
                    rows0_v[r, pl.ds(j * 16, 16)] = jnp.zeros(
                        (16,), jnp.float32)
                return carry
            lax.fori_loop(0, chunk, zrow, 0)
            zbase = sid * zrows
            for t in range(nzfull):
                pltpu.sync_copy(rows0_v,
                                acc_sh.at[pl.ds(zbase + t * chunk, chunk), :])
            if zrem:
                pltpu.sync_copy(
                    rows0_v.at[pl.ds(0, zrem), :],
                    acc_sh.at[pl.ds(zbase + nzfull * chunk, zrem), :])
            if const_ones:
                def orow(r, carry):
                    for j in range(F // 16):
                        rows0_v[r, pl.ds(j * 16, 16)] = jnp.ones(
                            (16,), jnp.float32)
                    return carry
                lax.fori_loop(0, chunk, orow, 0)
            plsc.subcore_barrier()

            if const_ones:
                def step(i, carry):
                    pltpu.sync_copy(rows0_v, acc_sh.at[dst_idx(i)],
                                    add=True)
                    return carry
                lax.fori_loop(0, nc_w, step, 0)
            else:
                base = cb * chunk

                def sload(i, buf):
                    pltpu.sync_copy(
                        src_hbm.at[pl.ds(base + i * chunk, chunk)], buf)

                def gather(idx_v, rows_v, sem):
                    pltpu.async_copy(g_hbm.at[idx_v], rows_v, sem)

                def gwait(rows_v, sem):
                    pltpu.make_async_copy(g_hbm.at[sidx0_v], rows_v,
                                          sem).wait()

                sload(0, sidx0_v)
                gather(sidx0_v, rows0_v, sem_a)
                sload(1, sidx1_v)

                def pair(j, carry):
                    i0 = 2 * j
                    gather(sidx1_v, rows1_v, sem_b)
                    d0 = dst_idx(i0)
                    gwait(rows0_v, sem_a)
                    sload(i0 + 2, sidx0_v)
                    pltpu.sync_copy(rows0_v, acc_sh.at[d0], add=True)
                    gather(sidx0_v, rows0_v, sem_a)
                    d1 = dst_idx(i0 + 1)
                    gwait(rows1_v, sem_b)
                    sload(i0 + 3, sidx1_v)
                    pltpu.sync_copy(rows1_v, acc_sh.at[d1], add=True)
                    return carry
                lax.fori_loop(0, nc_w // 2, pair, 0)
                gwait(rows0_v, sem_a)

            plsc.subcore_barrier()
            pltpu.sync_copy(acc_sh.at[pl.ds(sid * orows, orows), :],
                            out_hbm.at[cid, pl.ds(sid * orows, orows), :])

        work()

    return body


_sc_degcnt = _sc_gather_scatter_add(_DC_IDX, 16, _DC_ROWS, _DC_ROWS, 128,
                                    const_ones=True)
_sc_aggr = {
    128: _sc_gather_scatter_add(_E_PAD, 128, _OUT_ROWS, _OUT_ROWS, 128,
                                split=(128, 32)),
    64: _sc_gather_scatter_add(_E_PAD, 64, _OUT_ROWS, _OUT_ROWS, 128,
                               split=(120, 40)),
    32: _sc_gather_scatter_add(_E_PAD, 32, _OUT_ROWS, _OUT_ROWS, 128,
                               split=(120, 40)),
    16: _sc_gather_scatter_add(_E_PAD, 16, _OUT_ROWS, _OUT_ROWS, 128,
                               split=(112, 48)),
}
_sc_pool = _sc_gather_scatter_add(_POOL_IDX, 16, 128, 128, 64)


def _tc_dis(d0, d1):
    def body(a_ref, b_ref, o_ref):
        o_ref[...] = lax.rsqrt(a_ref[..., :1] + b_ref[..., :1] + 1.0)
    return pl.pallas_call(
        body,
        grid=(N // _BR,),
        in_specs=[pl.BlockSpec((_BR, 16), lambda i: (i, 0)),
                  pl.BlockSpec((_BR, 16), lambda i: (i, 0))],
        out_specs=pl.BlockSpec((_BR, 1), lambda i: (i, 0)),
        out_shape=jax.ShapeDtypeStruct((N, 1), jnp.float32),
    )(d0, d1)


def _tc_first(x, W, dis):
    F_in, F_out = W.shape

    def body(x_ref, w_ref, d_ref, o_ref):
        o_ref[...] = d_ref[...] * jnp.dot(
            x_ref[...], w_ref[...], preferred_element_type=jnp.float32)
    return pl.pallas_call(
        body,
        grid=(N // _BR,),
        in_specs=[pl.BlockSpec((_BR, F_in), lambda i: (i, 0)),
                  pl.BlockSpec((F_in, F_out), lambda i: (0, 0)),
                  pl.BlockSpec((_BR, 1), lambda i: (i, 0))],
        out_specs=pl.BlockSpec((_BR, F_out), lambda i: (i, 0)),
        out_shape=jax.ShapeDtypeStruct((N, F_out), jnp.float32),
    )(x, W, dis)


def _tc_mid(p0, p1, g, dis, b, W):
    F_in, F_out = W.shape

    def body(p0_ref, p1_ref, g_ref, d_ref, b_ref, w_ref, o_ref):
        a = jnp.maximum(
            d_ref[...] * (p0_ref[...] + p1_ref[...] + g_ref[...]) + b_ref[...],
            0.0)
        o_ref[...] = d_ref[...] * jnp.dot(
            a, w_ref[...], preferred_element_type=jnp.float32)
    return pl.pallas_call(
        body,
        grid=(N // _BR,),
        in_specs=[pl.BlockSpec((_BR, F_in), lambda i: (i, 0)),
                  pl.BlockSpec((_BR, F_in), lambda i: (i, 0)),
                  pl.BlockSpec((_BR, F_in), lambda i: (i, 0)),
                  pl.BlockSpec((_BR, 1), lambda i: (i, 0)),
                  pl.BlockSpec((1, F_in), lambda i: (0, 0)),
                  pl.BlockSpec((F_in, F_out), lambda i: (0, 0))],
        out_specs=pl.BlockSpec((_BR, F_out), lambda i: (i, 0)),
        out_shape=jax.ShapeDtypeStruct((N, F_out), jnp.float32),
    )(p0, p1, g, dis, b, W)


def _tc_mid1(p, g, dis, b, W):
    F_in, F_out = W.shape

    def body(p_ref, g_ref, d_ref, b_ref, w_ref, o_ref):
        a = jnp.maximum(
            d_ref[...] * (p_ref[...] + g_ref[...]) + b_ref[...], 0.0)
        o_ref[...] = d_ref[...] * jnp.dot(
            a, w_ref[...], preferred_element_type=jnp.float32)
    return pl.pallas_call(
        body,
        grid=(N // _BR,),
        in_specs=[pl.BlockSpec((_BR, F_in), lambda i: (i, 0)),
                  pl.BlockSpec((_BR, F_in), lambda i: (i, 0)),
                  pl.BlockSpec((_BR, 1), lambda i: (i, 0)),
                  pl.BlockSpec((1, F_in), lambda i: (0, 0)),
                  pl.BlockSpec((F_in, F_out), lambda i: (0, 0))],
        out_specs=pl.BlockSpec((_BR, F_out), lambda i: (i, 0)),
        out_shape=jax.ShapeDtypeStruct((N, F_out), jnp.float32),
    )(p, g, dis, b, W)


def _tc_post(p0, p1, g, dis, b):
    F = g.shape[1]

    def body(p0_ref, p1_ref, g_ref, d_ref, b_ref, o_ref):
        o_ref[...] = jnp.maximum(
            d_ref[...] * (p0_ref[...] + p1_ref[...] + g_ref[...]) + b_ref[...],
            0.0)
    return pl.pallas_call(
        body,
        grid=(N // _BR,),
        in_specs=[pl.BlockSpec((_BR, F), lambda i: (i, 0)),
                  pl.BlockSpec((_BR, F), lambda i: (i, 0)),
                  pl.BlockSpec((_BR, F), lambda i: (i, 0)),
                  pl.BlockSpec((_BR, 1), lambda i: (i, 0)),
                  pl.BlockSpec((1, F), lambda i: (0, 0))],
        out_specs=pl.BlockSpec((_BR, F), lambda i: (i, 0)),
        out_shape=jax.ShapeDtypeStruct((N, F), jnp.float32),
    )(p0, p1, g, dis, b)


def _tc_head(s0, s1, c0, c1, Wp, bp):
    def body(s0_ref, s1_ref, c0_ref, c1_ref, w_ref, b_ref, o_ref):
        s = s0_ref[...] + s1_ref[...]
        cnt = jnp.maximum(c0_ref[...] + c1_ref[...], 1.0)
        o_ref[...] = jnp.dot(
            s, w_ref[...], preferred_element_type=jnp.float32) / cnt + b_ref[...]
    return pl.pallas_call(
        body,
        out_shape=jax.ShapeDtypeStruct((G, Wp.shape[1]), jnp.float32),
    )(s0, s1, c0, c1, Wp, bp)


def kernel(x, edges, batch, W1, b1, W2, b2, W3, b3, W4, b4, Wp, bp):
    src, dst = edges[0], edges[1]

    ones16 = jnp.ones((8, 16), jnp.float32)
    dc_src = jnp.zeros((8,), jnp.int32)
    dc_dst = jnp.concatenate([
        dst,
        batch + _CNT_OFF,
        jnp.full((_DC_IDX - E - N,), _DISCARD, jnp.int32),
    ]).reshape(-1, 128)
    dc = _sc_degcnt(ones16, dc_src, dc_dst)
    dis = _tc_dis(dc[0], dc[1])

    e_src = jnp.concatenate([src, jnp.zeros((_E_PAD + 512 - E,), jnp.int32)])
    e_dst128 = jnp.concatenate([
        dst, jnp.full((_E_PAD + 256 * 128 - E,), _DISCARD, jnp.int32),
    ]).reshape(-1, 128)

    g1 = _tc_first(x, W1, dis)
    a1 = _sc_aggr[128](g1, e_src, e_dst128)
    g2 = _tc_mid(a1[0], a1[1], g1, dis, b1.reshape(1, -1), W2)
    a2 = _sc_aggr[64](g2, e_src, e_dst128)
    g3 = _tc_mid(a2[0], a2[1], g2, dis, b2.reshape(1, -1), W3)
    a3 = _sc_aggr[32](g3, e_src, e_dst128)
    g4 = _tc_mid(a3[0], a3[1], g3, dis, b3.reshape(1, -1), W4)
    a4 = _sc_aggr[16](g4, e_src, e_dst128)
    h4 = _tc_post(a4[0], a4[1], g4, dis, b4.reshape(1, -1))

    p_src = jnp.concatenate([
        jnp.arange(N, dtype=jnp.int32),
        jnp.zeros((_POOL_IDX + 128 - N,), jnp.int32),
    ])
    p_dst = jnp.concatenate([
        batch,
        jnp.full((_POOL_IDX - N,), G, jnp.int32),
    ]).reshape(-1, 64)
    pooled = _sc_pool(h4, p_src, p_dst)

    c0 = dc[0, _CNT_OFF:_CNT_OFF + G, :1]
    c1 = dc[1, _CNT_OFF:_CNT_OFF + G, :1]
    return _tc_head(pooled[0, :G], pooled[1, :G], c0, c1, Wp,
                    bp.reshape(1, -1))
